# baseline (speedup 1.0000x reference)
import jax
import jax.numpy as jnp
from jax.experimental import pallas as pl
from jax.experimental.pallas import tpu as pltpu

N = 4096
D = 256
BM = 512
BK = 512
NI = N // BM
NK = N // BK
BS = 512


def _support_body(x_ref, w_ref, out_ref):
    out_ref[...] = jnp.dot(x_ref[...], w_ref[...],
                           preferred_element_type=jnp.float32)


def _spmm_body(sup_ref, adj_ref, out_ref):
    k = pl.program_id(1)
    partial = jnp.dot(
        adj_ref[...],
        sup_ref[pl.ds(k * BK, BK), :],
        preferred_element_type=jnp.float32,
    )

    @pl.when(k == 0)
    def _init():
        out_ref[...] = partial

    @pl.when(k > 0)
    def _accum():
        out_ref[...] += partial


@jax.jit
def kernel(input, adj, W):
    support = pl.pallas_call(
        _support_body,
        grid=(N // BS,),
        in_specs=[
            pl.BlockSpec((BS, D), lambda i: (i, 0)),
            pl.BlockSpec((D, D), lambda i: (0, 0)),
        ],
        out_specs=pl.BlockSpec((BS, D), lambda i: (i, 0)),
        out_shape=jax.ShapeDtypeStruct((N, D), jnp.float32),
        compiler_params=pltpu.CompilerParams(
            dimension_semantics=("parallel",),
        ),
    )(input, W)

    return pl.pallas_call(
        _spmm_body,
        grid=(NI, NK),
        in_specs=[
            pl.BlockSpec((N, D), lambda i, k: (0, 0)),
            pl.BlockSpec((BM, BK), lambda i, k: (i, k)),
        ],
        out_specs=pl.BlockSpec((BM, D), lambda i, k: (i, 0)),
        out_shape=jax.ShapeDtypeStruct((N, D), jnp.float32),
        compiler_params=pltpu.CompilerParams(
            dimension_semantics=("parallel", "arbitrary"),
        ),
    )(support, adj)


# speedup vs baseline: 2.1975x; 2.1975x over previous
import jax
import jax.numpy as jnp
from jax.experimental import pallas as pl
from jax.experimental.pallas import tpu as pltpu

N = 4096
D = 256
BM = 1024
BK = 1024
NI = N // BM
NK = N // BK


def _gcn_body(x_ref, adj_ref, w_ref, out_ref, support_ref):
    i = pl.program_id(0)
    k = pl.program_id(1)

    @pl.when(i == 0)
    def _compute_support():
        support_ref[pl.ds(k * BK, BK), :] = jnp.dot(
            x_ref[...], w_ref[...], preferred_element_type=jnp.float32
        )

    partial = jnp.dot(
        adj_ref[...],
        support_ref[pl.ds(k * BK, BK), :],
        preferred_element_type=jnp.float32,
    )

    @pl.when(k == 0)
    def _init():
        out_ref[...] = partial

    @pl.when(k > 0)
    def _accum():
        out_ref[...] += partial


@jax.jit
def kernel(input, adj, W):
    grid = (NI, NK)
    return pl.pallas_call(
        _gcn_body,
        grid=grid,
        in_specs=[
            pl.BlockSpec((BK, D), lambda i, k: (jnp.where(i == 0, k, NK - 1), 0)),
            pl.BlockSpec((BM, BK), lambda i, k: (i, k)),
            pl.BlockSpec((D, D), lambda i, k: (0, 0)),
        ],
        out_specs=pl.BlockSpec((BM, D), lambda i, k: (i, 0)),
        out_shape=jax.ShapeDtypeStruct((N, D), jnp.float32),
        scratch_shapes=[pltpu.VMEM((N, D), jnp.float32)],
        compiler_params=pltpu.CompilerParams(
            dimension_semantics=("arbitrary", "arbitrary"),
        ),
    )(input, adj, W)


# speedup vs baseline: 2.4872x; 1.1318x over previous
import jax
import jax.numpy as jnp
from jax.experimental import pallas as pl
from jax.experimental.pallas import tpu as pltpu

N = 4096
D = 256
BM = 2048
BK = 1024
NI = N // BM
NK = N // BK


def _gcn_body(x_ref, adj_ref, w_ref, out_ref, support_ref):
    i = pl.program_id(0)
    k = pl.program_id(1)

    @pl.when(i == 0)
    def _compute_support():
        support_ref[pl.ds(k * BK, BK), :] = jnp.dot(
            x_ref[...], w_ref[...], preferred_element_type=jnp.float32
        )

    partial = jnp.dot(
        adj_ref[...],
        support_ref[pl.ds(k * BK, BK), :],
        preferred_element_type=jnp.float32,
    )

    @pl.when(k == 0)
    def _init():
        out_ref[...] = partial

    @pl.when(k > 0)
    def _accum():
        out_ref[...] += partial


@jax.jit
def kernel(input, adj, W):
    grid = (NI, NK)
    return pl.pallas_call(
        _gcn_body,
        grid=grid,
        in_specs=[
            pl.BlockSpec((BK, D), lambda i, k: (jnp.where(i == 0, k, NK - 1), 0)),
            pl.BlockSpec((BM, BK), lambda i, k: (i, k)),
            pl.BlockSpec((D, D), lambda i, k: (0, 0)),
        ],
        out_specs=pl.BlockSpec((BM, D), lambda i, k: (i, 0)),
        out_shape=jax.ShapeDtypeStruct((N, D), jnp.float32),
        scratch_shapes=[pltpu.VMEM((N, D), jnp.float32)],
        compiler_params=pltpu.CompilerParams(
            dimension_semantics=("arbitrary", "arbitrary"),
        ),
    )(input, adj, W)


# speedup vs baseline: 2.5366x; 1.0199x over previous
import jax
import jax.numpy as jnp
from jax.experimental import pallas as pl
from jax.experimental.pallas import tpu as pltpu

N = 4096
D = 256
BM = 512
BK = 4096
NI = N // BM
NK = N // BK


def _gcn_body(x_ref, adj_ref, w_ref, out_ref, support_ref):
    i = pl.program_id(0)
    k = pl.program_id(1)

    @pl.when(i == 0)
    def _compute_support():
        support_ref[pl.ds(k * BK, BK), :] = jnp.dot(
            x_ref[...], w_ref[...], preferred_element_type=jnp.float32
        )

    partial = jnp.dot(
        adj_ref[...],
        support_ref[pl.ds(k * BK, BK), :],
        preferred_element_type=jnp.float32,
    )

    @pl.when(k == 0)
    def _init():
        out_ref[...] = partial

    @pl.when(k > 0)
    def _accum():
        out_ref[...] += partial


@jax.jit
def kernel(input, adj, W):
    grid = (NI, NK)
    return pl.pallas_call(
        _gcn_body,
        grid=grid,
        in_specs=[
            pl.BlockSpec((BK, D), lambda i, k: (jnp.where(i == 0, k, NK - 1), 0)),
            pl.BlockSpec((BM, BK), lambda i, k: (i, k)),
            pl.BlockSpec((D, D), lambda i, k: (0, 0)),
        ],
        out_specs=pl.BlockSpec((BM, D), lambda i, k: (i, 0)),
        out_shape=jax.ShapeDtypeStruct((N, D), jnp.float32),
        scratch_shapes=[pltpu.VMEM((N, D), jnp.float32)],
        compiler_params=pltpu.CompilerParams(
            dimension_semantics=("arbitrary", "arbitrary"),
        ),
    )(input, adj, W)
